# Initial kernel scaffold; baseline (speedup 1.0000x reference)
import jax
import jax.numpy as jnp
from jax.experimental import pallas as pl


def kernel(x, res_w1, res_b1, res_w2, res_b2, id_w, id_b, down_w):
    raise NotImplementedError("write your pallas kernel here")


# speedup vs baseline: 13.3420x; 13.3420x over previous
import functools

import jax
import jax.numpy as jnp
from jax.experimental import pallas as pl
from jax.experimental.pallas import tpu as pltpu

_VMEM_LIMIT = 48 * 1024 * 1024


def _leaky(v, slope):
    return jnp.where(v >= 0.0, v, slope * v)


def _fused_kernel(xp_ref, w1_ref, b1_ref, w2_ref, b2_ref, wi_ref, bi_ref,
                  wd_ref, out_ref, down_ref, h1_ref, op_ref, *, H, W, slope):
    cin = xp_ref.shape[3]
    cout = out_ref.shape[3]
    Ho, Wo = H // 2, W // 2

    acc = jnp.zeros((H * W, cout), jnp.float32)
    for dy in range(3):
        for dx in range(3):
            patch = xp_ref[0, dy:dy + H, dx:dx + W, :].reshape(H * W, cin)
            acc = acc + jnp.dot(patch, w1_ref[dy * 3 + dx],
                                preferred_element_type=jnp.float32)
    h1 = _leaky(acc + b1_ref[...], slope).astype(jnp.bfloat16)

    h1_ref[...] = jnp.zeros((H + 2, W + 2, cout), jnp.bfloat16)
    h1_ref[1:H + 1, 1:W + 1, :] = h1.reshape(H, W, cout)

    acc2 = jnp.zeros((H * W, cout), jnp.float32)
    for dy in range(3):
        for dx in range(3):
            patch = h1_ref[dy:dy + H, dx:dx + W, :].reshape(H * W, cout)
            acc2 = acc2 + jnp.dot(patch, w2_ref[dy * 3 + dx],
                                  preferred_element_type=jnp.float32)
    res = _leaky(acc2 + b2_ref[...], slope)
    xin = xp_ref[0, 1:H + 1, 1:W + 1, :].reshape(H * W, cin)
    ident = jnp.dot(xin, wi_ref[...],
                    preferred_element_type=jnp.float32) + bi_ref[...]
    outv = res + 2.0 * ident
    out_ref[0] = outv.reshape(H, W, cout)

    op_ref[...] = jnp.zeros((H + 2, W + 2, cout), jnp.bfloat16)
    op_ref[1:H + 1, 1:W + 1, :] = outv.astype(jnp.bfloat16).reshape(H, W, cout)

    acc3 = jnp.zeros((Ho * Wo, cout), jnp.float32)
    for ky in range(4):
        p = ky % 2
        s = ky - p
        rows = op_ref[s:s + H, :, :].reshape(Ho, 2, W + 2, cout)[:, p]
        for kx in range(4):
            q = kx % 2
            t = kx - q
            cols = rows[:, t:t + W, :].reshape(Ho, Wo, 2, cout)[:, :, q]
            acc3 = acc3 + jnp.dot(cols.reshape(Ho * Wo, cout),
                                  wd_ref[ky * 4 + kx],
                                  preferred_element_type=jnp.float32)
    down_ref[0] = acc3.reshape(Ho, Wo, cout)


def kernel(x, res_w1, res_b1, res_w2, res_b2, id_w, id_b, down_w):
    N, cin, H, W = x.shape
    cout = res_w1.shape[0]
    Ho, Wo = H // 2, W // 2
    slope = 0.2

    xn = jnp.transpose(x, (0, 2, 3, 1))
    xp = jnp.pad(xn, ((0, 0), (1, 1), (1, 1), (0, 0))).astype(jnp.bfloat16)

    def conv_w(w):
        co, ci = w.shape[0], w.shape[1]
        k = w.shape[2] * w.shape[3]
        return jnp.transpose(w, (2, 3, 1, 0)).reshape(k, ci, co).astype(jnp.bfloat16)

    w1 = conv_w(res_w1)
    w2 = conv_w(res_w2)
    wd = conv_w(down_w)
    wi = id_w.reshape(cout, cin).T.astype(jnp.bfloat16)
    b1 = res_b1.reshape(1, cout).astype(jnp.float32)
    b2 = res_b2.reshape(1, cout).astype(jnp.float32)
    bi = id_b.reshape(1, cout).astype(jnp.float32)

    out, down = pl.pallas_call(
        functools.partial(_fused_kernel, H=H, W=W, slope=slope),
        out_shape=[
            jax.ShapeDtypeStruct((N, H, W, cout), jnp.float32),
            jax.ShapeDtypeStruct((N, Ho, Wo, cout), jnp.float32),
        ],
        grid_spec=pltpu.PrefetchScalarGridSpec(
            num_scalar_prefetch=0,
            grid=(N,),
            in_specs=[
                pl.BlockSpec((1, H + 2, W + 2, cin), lambda n: (n, 0, 0, 0)),
                pl.BlockSpec((9, cin, cout), lambda n: (0, 0, 0)),
                pl.BlockSpec((1, cout), lambda n: (0, 0)),
                pl.BlockSpec((9, cout, cout), lambda n: (0, 0, 0)),
                pl.BlockSpec((1, cout), lambda n: (0, 0)),
                pl.BlockSpec((cin, cout), lambda n: (0, 0)),
                pl.BlockSpec((1, cout), lambda n: (0, 0)),
                pl.BlockSpec((16, cout, cout), lambda n: (0, 0, 0)),
            ],
            out_specs=[
                pl.BlockSpec((1, H, W, cout), lambda n: (n, 0, 0, 0)),
                pl.BlockSpec((1, Ho, Wo, cout), lambda n: (n, 0, 0, 0)),
            ],
            scratch_shapes=[
                pltpu.VMEM((H + 2, W + 2, cout), jnp.bfloat16),
                pltpu.VMEM((H + 2, W + 2, cout), jnp.bfloat16),
            ],
        ),
        compiler_params=pltpu.CompilerParams(
            dimension_semantics=("parallel",), vmem_limit_bytes=_VMEM_LIMIT),
    )(xp, w1, b1, w2, b2, wi, bi, wd)

    return jnp.transpose(down, (0, 3, 1, 2)), jnp.transpose(out, (0, 3, 1, 2))
